# baseline (speedup 1.0000x reference)
import jax
import jax.numpy as jnp
from jax.experimental import pallas as pl


def _pos_kernel(xe_ref, yz_y_ref, yz_z_ref, out_ref):
    x = xe_ref[...]
    yz = yz_y_ref[...] + yz_z_ref[...]
    out_ref[...] = x[None, :, :, None] + yz[None, :, None, :]


def kernel(features, x_embed, y_embed, z_embed):
    b = features.shape[0]
    nx, ny, nz = features.shape[2], features.shape[3], features.shape[4]
    d = x_embed.shape[1]

    xeT = x_embed[:nx].T
    ye_rep = jnp.repeat(y_embed[:ny].T, nz, axis=1)
    ze_til = jnp.tile(z_embed[:nz].T, (1, ny))

    D = 32
    grid = (b, d // D)

    out = pl.pallas_call(
        _pos_kernel,
        grid=grid,
        in_specs=[
            pl.BlockSpec((D, nx), lambda i, j: (j, 0)),
            pl.BlockSpec((D, ny * nz), lambda i, j: (j, 0)),
            pl.BlockSpec((D, ny * nz), lambda i, j: (j, 0)),
        ],
        out_specs=pl.BlockSpec((1, D, nx, ny * nz), lambda i, j: (i, j, 0, 0)),
        out_shape=jax.ShapeDtypeStruct((b, d, nx, ny * nz), jnp.float32),
    )(xeT, ye_rep, ze_til)

    return out.reshape(b, d, nx, ny, nz)


# speedup vs baseline: 1.0161x; 1.0161x over previous
import jax
import jax.numpy as jnp
from jax.experimental import pallas as pl
from jax.experimental.pallas import tpu as pltpu


def _pos_kernel(xe_ref, yz_y_ref, yz_z_ref, out_ref):
    x = xe_ref[...]
    yz = yz_y_ref[...] + yz_z_ref[...]
    out_ref[...] = x[None, :, :, None] + yz[None, :, None, :]


def kernel(features, x_embed, y_embed, z_embed):
    b = features.shape[0]
    nx, ny, nz = features.shape[2], features.shape[3], features.shape[4]
    d = x_embed.shape[1]

    xeT = x_embed[:nx].T
    ye_rep = jnp.repeat(y_embed[:ny].T, nz, axis=1)
    ze_til = jnp.tile(z_embed[:nz].T, (1, ny))

    D = 64
    grid = (b, d // D)

    out = pl.pallas_call(
        _pos_kernel,
        grid=grid,
        in_specs=[
            pl.BlockSpec((D, nx), lambda i, j: (j, 0)),
            pl.BlockSpec((D, ny * nz), lambda i, j: (j, 0)),
            pl.BlockSpec((D, ny * nz), lambda i, j: (j, 0)),
        ],
        out_specs=pl.BlockSpec((1, D, nx, ny * nz), lambda i, j: (i, j, 0, 0)),
        out_shape=jax.ShapeDtypeStruct((b, d, nx, ny * nz), jnp.float32),
        compiler_params=pltpu.CompilerParams(
            dimension_semantics=("parallel", "parallel"),
        ),
    )(xeT, ye_rep, ze_til)

    return out.reshape(b, d, nx, ny, nz)


# speedup vs baseline: 1.0301x; 1.0138x over previous
import jax
import jax.numpy as jnp
from jax.experimental import pallas as pl
from jax.experimental.pallas import tpu as pltpu

D = 32


def _pos_kernel_mdma(nsteps, b, xe_ref, yrep_ref, ztil_ref, out_ref, scratch, sems):
    j = pl.program_id(0)
    slot = j % 2

    def copies(s, step):
        return [
            pltpu.make_async_copy(
                scratch.at[s],
                out_ref.at[i, pl.ds(step * D, D)],
                sems.at[s, i],
            )
            for i in range(b)
        ]

    @pl.when(j >= 2)
    def _():
        for c in copies(slot, j - 2):
            c.wait()

    x = xe_ref[pl.ds(j * D, D), :]
    yz = yrep_ref[pl.ds(j * D, D), :] + ztil_ref[pl.ds(j * D, D), :]
    scratch[slot] = x[:, :, None] + yz[:, None, :]

    for c in copies(slot, j):
        c.start()

    @pl.when(j == nsteps - 1)
    def _():
        for c in copies(1 - slot, j - 1):
            c.wait()
        for c in copies(slot, j):
            c.wait()


def kernel(features, x_embed, y_embed, z_embed):
    b = features.shape[0]
    nx, ny, nz = features.shape[2], features.shape[3], features.shape[4]
    d = x_embed.shape[1]
    nyz = ny * nz

    xeT = x_embed[:nx].T
    ye_rep = jnp.repeat(y_embed[:ny].T, nz, axis=1)
    ze_til = jnp.tile(z_embed[:nz].T, (1, ny))

    nsteps = d // D
    import functools
    body = functools.partial(_pos_kernel_mdma, nsteps, b)

    out = pl.pallas_call(
        body,
        grid=(nsteps,),
        in_specs=[
            pl.BlockSpec(memory_space=pltpu.MemorySpace.VMEM),
            pl.BlockSpec(memory_space=pltpu.MemorySpace.VMEM),
            pl.BlockSpec(memory_space=pltpu.MemorySpace.VMEM),
        ],
        out_specs=pl.BlockSpec(memory_space=pltpu.MemorySpace.HBM),
        out_shape=jax.ShapeDtypeStruct((b, d, nx, nyz), jnp.float32),
        scratch_shapes=[
            pltpu.VMEM((2, D, nx, nyz), jnp.float32),
            pltpu.SemaphoreType.DMA((2, b)),
        ],
        compiler_params=pltpu.CompilerParams(
            dimension_semantics=("arbitrary",),
        ),
    )(xeT, ye_rep, ze_til)

    return out.reshape(b, d, nx, ny, nz)
